# baseline (speedup 1.0000x reference)
import functools

import jax
import jax.numpy as jnp
from jax import lax
from jax.experimental import pallas as pl
from jax.experimental.pallas import tpu as pltpu
from jax.experimental.pallas import tpu_sc as plsc

N = 10000
E = 320000
D = 128

NC = 2
NS = 16
L = 16
NW = NC * NS

NP = 10240
NPR = NP // NS
K = 16
NCHUNK = 648
EPW = NCHUNK * K
E_EXT = NW * EPW
NBUF = 8
DEG_W = 8


def _sc_mesh():
    return plsc.VectorSubcoreMesh(core_axis_name="c", subcore_axis_name="s")


@functools.partial(
    pl.kernel,
    out_type=jax.ShapeDtypeStruct((NC, NP), jnp.float32),
    mesh=_sc_mesh(),
    scratch_types=[
        pltpu.VMEM((EPW,), jnp.int32),
        pltpu.VMEM((EPW,), jnp.float32),
        pltpu.VMEM_SHARED((NP,), jnp.float32),
        pltpu.SemaphoreType.DMA,
    ],
    compiler_params=pltpu.CompilerParams(needs_layout_passes=False),
)
def _deg_sc(col3_hbm, ew3_hbm, zeros_hbm, out_hbm, cidx_all, ew_all, acc_sh, dsem):
    c = lax.axis_index("c")
    s = lax.axis_index("s")
    wid = s * NC + c
    pltpu.sync_copy(col3_hbm.at[wid], cidx_all)
    pltpu.sync_copy(ew3_hbm.at[wid], ew_all)
    pltpu.sync_copy(zeros_hbm.at[pl.ds(s * NPR, NPR)], acc_sh.at[pl.ds(s * NPR, NPR)])
    plsc.subcore_barrier()

    def chunk(kk, carry):
        pltpu.async_copy(ew_all.at[pl.ds(kk * K, K)],
                         acc_sh.at[cidx_all.at[pl.ds(kk * K, K)]], dsem, add=True)

        @pl.when(kk >= DEG_W)
        def _():
            kp = kk - DEG_W
            pltpu.make_async_copy(ew_all.at[pl.ds(kp * K, K)],
                                  acc_sh.at[cidx_all.at[pl.ds(kp * K, K)]],
                                  dsem).wait()

        return carry

    lax.fori_loop(0, NCHUNK, chunk, jnp.int32(0))

    def drain(w, carry):
        kk = NCHUNK - DEG_W + w
        pltpu.make_async_copy(ew_all.at[pl.ds(kk * K, K)],
                              acc_sh.at[cidx_all.at[pl.ds(kk * K, K)]],
                              dsem).wait()
        return carry

    lax.fori_loop(0, DEG_W, drain, jnp.int32(0))
    plsc.subcore_barrier()
    pltpu.sync_copy(acc_sh.at[pl.ds(s * NPR, NPR)], out_hbm.at[c, pl.ds(s * NPR, NPR)])


@functools.partial(
    pl.kernel,
    out_type=jax.ShapeDtypeStruct((NC, NP, D), jnp.float32),
    mesh=_sc_mesh(),
    scratch_types=[
        pltpu.VMEM((EPW,), jnp.int32),
        pltpu.VMEM((EPW,), jnp.int32),
        pltpu.VMEM((EPW,), jnp.float32),
        pltpu.VMEM((K, D), jnp.float32),
        pltpu.VMEM((K, D), jnp.float32),
        pltpu.VMEM((K, D), jnp.float32),
        pltpu.VMEM((K, D), jnp.float32),
        pltpu.VMEM((K, D), jnp.float32),
        pltpu.VMEM((K, D), jnp.float32),
        pltpu.VMEM((K, D), jnp.float32),
        pltpu.VMEM((K, D), jnp.float32),
        pltpu.VMEM_SHARED((NP, D), jnp.float32),
    ] + [pltpu.SemaphoreType.DMA] * 16,
    compiler_params=pltpu.CompilerParams(needs_layout_passes=False),
)
def _prop_sc(row3_hbm, col3_hbm, ew3_hbm, xw_hbm, zeros_hbm, out_hbm,
             ridx_all, cidx_all, ew_all, *bufs_and_sems):
    rows = bufs_and_sems[:NBUF]
    acc_sh = bufs_and_sems[NBUF]
    gsem = bufs_and_sems[NBUF + 1:2 * NBUF + 1]
    ssem = bufs_and_sems[2 * NBUF + 1:3 * NBUF + 1]
    c = lax.axis_index("c")
    s = lax.axis_index("s")
    wid = s * NC + c

    pltpu.sync_copy(row3_hbm.at[wid], ridx_all)
    pltpu.sync_copy(col3_hbm.at[wid], cidx_all)
    pltpu.sync_copy(ew3_hbm.at[wid], ew_all)
    pltpu.sync_copy(zeros_hbm.at[pl.ds(s * NPR, NPR), :],
                    acc_sh.at[pl.ds(s * NPR, NPR), :])
    plsc.subcore_barrier()

    for i in range(NBUF - 1):
        pltpu.async_copy(xw_hbm.at[ridx_all.at[pl.ds(i * K, K)]],
                         rows[i], gsem[i])

    def step(t, carry):
        for b in range(NBUF):
            kk = NBUF * t + b
            rb, gb, sb = rows[b], gsem[b], ssem[b]
            bo = (b + NBUF - 1) % NBUF

            @pl.when(kk >= 1)
            def _():
                pltpu.make_async_copy(rows[bo],
                                      acc_sh.at[cidx_all.at[pl.ds((kk - 1) * K, K)]],
                                      ssem[bo]).wait()

            @pl.when(kk + NBUF - 1 < NCHUNK)
            def _():
                pltpu.async_copy(
                    xw_hbm.at[ridx_all.at[pl.ds((kk + NBUF - 1) * K, K)]],
                    rows[bo], gsem[bo])

            pltpu.make_async_copy(xw_hbm.at[ridx_all.at[pl.ds(kk * K, K)]],
                                  rb, gb).wait()

            def scale_group(g, cy):
                wv = ew_all[pl.ds(kk * K + g * L, L)]
                ebase = g * L
                for j in range(L):
                    bv = jnp.full((L,), wv[j], dtype=jnp.float32)
                    for d in range(D // L):
                        fsl = pl.ds(d * L, L)
                        rb[ebase + j, fsl] = rb[ebase + j, fsl] * bv
                return cy

            lax.fori_loop(0, K // L, scale_group, carry)

            pltpu.async_copy(rb, acc_sh.at[cidx_all.at[pl.ds(kk * K, K)]],
                             sb, add=True)

        return carry

    lax.fori_loop(0, NCHUNK // NBUF, step, jnp.int32(0))
    pltpu.make_async_copy(rows[(NCHUNK - 1) % NBUF],
                          acc_sh.at[cidx_all.at[pl.ds((NCHUNK - 1) * K, K)]],
                          ssem[(NCHUNK - 1) % NBUF]).wait()
    plsc.subcore_barrier()
    pltpu.sync_copy(acc_sh.at[pl.ds(s * NPR, NPR), :],
                    out_hbm.at[c, pl.ds(s * NPR, NPR), :])


def _tc1_body(degp_ref, x_ref, w_ref, dis_ref, y_ref):
    deg = degp_ref[0] + degp_ref[1]
    dis = jnp.where(deg > 0.0, lax.rsqrt(deg), 0.0)
    dis_ref[...] = dis
    y_ref[...] = jnp.dot(x_ref[...] * dis[:N], w_ref[...],
                         preferred_element_type=jnp.float32)


def _tc2_body(p_ref, dis_ref, b_ref, w_ref, y_ref):
    disn = dis_ref[:N]
    h = jnp.maximum(disn * (p_ref[0, :N, :] + p_ref[1, :N, :]) + b_ref[...],
                    0.0)
    y_ref[...] = jnp.dot(disn * h, w_ref[...],
                         preferred_element_type=jnp.float32)


def _tc3_body(p_ref, dis_ref, b_ref, out_ref):
    out_ref[...] = (dis_ref[:N] * (p_ref[0, :N, :] + p_ref[1, :N, :])
                    + b_ref[...])


_tc1 = pl.pallas_call(
    _tc1_body,
    out_shape=(jax.ShapeDtypeStruct((NP, 1), jnp.float32),
               jax.ShapeDtypeStruct((N, D), jnp.float32)),
)
_tc2 = pl.pallas_call(
    _tc2_body,
    out_shape=jax.ShapeDtypeStruct((N, D), jnp.float32),
)
_tc3 = pl.pallas_call(
    _tc3_body,
    out_shape=jax.ShapeDtypeStruct((N, D), jnp.float32),
)


def kernel(x, edge_index, edge_weight, W1, b1, W2, b2):
    row = edge_index[0].astype(jnp.int32)
    col = edge_index[1].astype(jnp.int32)
    ew = edge_weight.astype(jnp.float32)

    loop_idx = jnp.arange(N, dtype=jnp.int32)
    npad = E_EXT - E - N
    zpad_i = jnp.zeros((npad,), jnp.int32)
    row_e = jnp.concatenate([row, loop_idx, zpad_i]).reshape(NW, EPW)
    col_e = jnp.concatenate([col, loop_idx, zpad_i]).reshape(NW, EPW)
    ew_e = jnp.concatenate([ew, jnp.ones((N,), jnp.float32),
                            jnp.zeros((npad,), jnp.float32)]).reshape(NW, EPW)

    zeros1 = jnp.zeros((NP,), jnp.float32)
    zerosR = jnp.zeros((NP, D), jnp.float32)

    degp = _deg_sc(col_e, ew_e, zeros1)
    dis, y1 = _tc1(degp.reshape(NC, NP, 1), x, W1)
    p1 = _prop_sc(row_e, col_e, ew_e, y1, zerosR)
    y2 = _tc2(p1, dis, b1.reshape(1, D), W2)
    p2 = _prop_sc(row_e, col_e, ew_e, y2, zerosR)
    out = _tc3(p2, dis, b2.reshape(1, D))
    return out


# speedup vs baseline: 1.1067x; 1.1067x over previous
import functools

import jax
import jax.numpy as jnp
from jax import lax
from jax.experimental import pallas as pl
from jax.experimental.pallas import tpu as pltpu
from jax.experimental.pallas import tpu_sc as plsc

N = 10000
E = 320000
D = 128

NC = 2
NS = 16
L = 16
NW = NC * NS

NP = 10240
NPR = NP // NS
K = 32
NCHUNK = 324
EPW = NCHUNK * K
E_EXT = NW * EPW
NBUF = 4
DEG_W = 8


def _sc_mesh():
    return plsc.VectorSubcoreMesh(core_axis_name="c", subcore_axis_name="s")


@functools.partial(
    pl.kernel,
    out_type=jax.ShapeDtypeStruct((NC, NP), jnp.float32),
    mesh=_sc_mesh(),
    scratch_types=[
        pltpu.VMEM((EPW,), jnp.int32),
        pltpu.VMEM((EPW,), jnp.float32),
        pltpu.VMEM_SHARED((NP,), jnp.float32),
        pltpu.SemaphoreType.DMA,
    ],
    compiler_params=pltpu.CompilerParams(needs_layout_passes=False),
)
def _deg_sc(col3_hbm, ew3_hbm, zeros_hbm, out_hbm, cidx_all, ew_all, acc_sh, dsem):
    c = lax.axis_index("c")
    s = lax.axis_index("s")
    wid = s * NC + c
    pltpu.async_copy(col3_hbm.at[wid], cidx_all, dsem)
    pltpu.async_copy(ew3_hbm.at[wid], ew_all, dsem)
    pltpu.async_copy(zeros_hbm.at[pl.ds(s * NPR, NPR)],
                     acc_sh.at[pl.ds(s * NPR, NPR)], dsem)
    pltpu.make_async_copy(col3_hbm.at[wid], cidx_all, dsem).wait()
    pltpu.make_async_copy(ew3_hbm.at[wid], ew_all, dsem).wait()
    pltpu.make_async_copy(zeros_hbm.at[pl.ds(s * NPR, NPR)],
                          acc_sh.at[pl.ds(s * NPR, NPR)], dsem).wait()
    plsc.subcore_barrier()

    def chunk(kk, carry):
        pltpu.async_copy(ew_all.at[pl.ds(kk * K, K)],
                         acc_sh.at[cidx_all.at[pl.ds(kk * K, K)]], dsem, add=True)

        @pl.when(kk >= DEG_W)
        def _():
            kp = kk - DEG_W
            pltpu.make_async_copy(ew_all.at[pl.ds(kp * K, K)],
                                  acc_sh.at[cidx_all.at[pl.ds(kp * K, K)]],
                                  dsem).wait()

        return carry

    lax.fori_loop(0, NCHUNK, chunk, jnp.int32(0))

    def drain(w, carry):
        kk = NCHUNK - DEG_W + w
        pltpu.make_async_copy(ew_all.at[pl.ds(kk * K, K)],
                              acc_sh.at[cidx_all.at[pl.ds(kk * K, K)]],
                              dsem).wait()
        return carry

    lax.fori_loop(0, DEG_W, drain, jnp.int32(0))
    plsc.subcore_barrier()
    pltpu.sync_copy(acc_sh.at[pl.ds(s * NPR, NPR)], out_hbm.at[c, pl.ds(s * NPR, NPR)])


@functools.partial(
    pl.kernel,
    out_type=jax.ShapeDtypeStruct((NC, NP, D), jnp.float32),
    mesh=_sc_mesh(),
    scratch_types=[
        pltpu.VMEM((EPW,), jnp.int32),
        pltpu.VMEM((EPW,), jnp.int32),
        pltpu.VMEM((EPW,), jnp.float32),
        pltpu.VMEM((K, D), jnp.float32),
        pltpu.VMEM((K, D), jnp.float32),
        pltpu.VMEM((K, D), jnp.float32),
        pltpu.VMEM((K, D), jnp.float32),
        pltpu.VMEM_SHARED((NP, D), jnp.float32),
        pltpu.SemaphoreType.DMA,
        pltpu.SemaphoreType.DMA,
        pltpu.SemaphoreType.DMA,
        pltpu.SemaphoreType.DMA,
        pltpu.SemaphoreType.DMA,
        pltpu.SemaphoreType.DMA,
        pltpu.SemaphoreType.DMA,
        pltpu.SemaphoreType.DMA,
    ],
    compiler_params=pltpu.CompilerParams(needs_layout_passes=False),
)
def _prop_sc(row3_hbm, col3_hbm, ew3_hbm, xw_hbm, out_hbm,
             ridx_all, cidx_all, ew_all, rows0, rows1, rows2, rows3, acc_sh,
             g0, g1, g2, g3, s0, s1, s2, s3):
    c = lax.axis_index("c")
    s = lax.axis_index("s")
    wid = s * NC + c
    rows = (rows0, rows1, rows2, rows3)
    gsem = (g0, g1, g2, g3)
    ssem = (s0, s1, s2, s3)

    pltpu.async_copy(row3_hbm.at[wid], ridx_all, g0)
    pltpu.async_copy(col3_hbm.at[wid], cidx_all, g1)
    pltpu.async_copy(ew3_hbm.at[wid], ew_all, g2)
    zv = jnp.zeros((L,), jnp.float32)
    for i in range(K):
        for d in range(D // L):
            rows[NBUF - 1][i, pl.ds(d * L, L)] = zv
    pltpu.make_async_copy(row3_hbm.at[wid], ridx_all, g0).wait()
    pltpu.make_async_copy(col3_hbm.at[wid], cidx_all, g1).wait()
    pltpu.make_async_copy(ew3_hbm.at[wid], ew_all, g2).wait()
    for r in range(NPR // K):
        pltpu.async_copy(rows[NBUF - 1],
                         acc_sh.at[pl.ds(s * NPR + r * K, K), :], s0)
    for i in range(NBUF - 1):
        pltpu.async_copy(xw_hbm.at[ridx_all.at[pl.ds(i * K, K)]],
                         rows[i], gsem[i])
    for r in range(NPR // K):
        pltpu.make_async_copy(rows[NBUF - 1],
                              acc_sh.at[pl.ds(s * NPR + r * K, K), :],
                              s0).wait()
    plsc.subcore_barrier()

    def step(t, carry):
        for b in range(NBUF):
            kk = NBUF * t + b
            rb, gb, sb = rows[b], gsem[b], ssem[b]
            bo = (b + NBUF - 1) % NBUF

            @pl.when(kk >= 1)
            def _():
                pltpu.make_async_copy(rows[bo],
                                      acc_sh.at[cidx_all.at[pl.ds((kk - 1) * K, K)]],
                                      ssem[bo]).wait()

            @pl.when(kk + NBUF - 1 < NCHUNK)
            def _():
                pltpu.async_copy(
                    xw_hbm.at[ridx_all.at[pl.ds((kk + NBUF - 1) * K, K)]],
                    rows[bo], gsem[bo])

            pltpu.make_async_copy(xw_hbm.at[ridx_all.at[pl.ds(kk * K, K)]],
                                  rb, gb).wait()

            def scale_group(g, cy):
                wv = ew_all[pl.ds(kk * K + g * L, L)]
                ebase = g * L
                for j in range(L):
                    bv = jnp.full((L,), wv[j], dtype=jnp.float32)
                    for d in range(D // L):
                        fsl = pl.ds(d * L, L)
                        rb[ebase + j, fsl] = rb[ebase + j, fsl] * bv
                return cy

            lax.fori_loop(0, K // L, scale_group, carry)

            pltpu.async_copy(rb, acc_sh.at[cidx_all.at[pl.ds(kk * K, K)]],
                             sb, add=True)

        return carry

    lax.fori_loop(0, NCHUNK // NBUF, step, jnp.int32(0))
    pltpu.make_async_copy(rows[(NCHUNK - 1) % NBUF],
                          acc_sh.at[cidx_all.at[pl.ds((NCHUNK - 1) * K, K)]],
                          ssem[(NCHUNK - 1) % NBUF]).wait()
    plsc.subcore_barrier()
    pltpu.sync_copy(acc_sh.at[pl.ds(s * NPR, NPR), :],
                    out_hbm.at[c, pl.ds(s * NPR, NPR), :])


def _tc1_body(degp_ref, x_ref, w_ref, dis_ref, y_ref):
    deg = degp_ref[0] + degp_ref[1]
    dis = jnp.where(deg > 0.0, lax.rsqrt(deg), 0.0)
    dis_ref[...] = dis
    y_ref[...] = jnp.dot(x_ref[...] * dis[:N], w_ref[...],
                         preferred_element_type=jnp.float32)


def _tc2_body(p_ref, dis_ref, b_ref, w_ref, y_ref):
    disn = dis_ref[:N]
    h = jnp.maximum(disn * (p_ref[0, :N, :] + p_ref[1, :N, :]) + b_ref[...],
                    0.0)
    y_ref[...] = jnp.dot(disn * h, w_ref[...],
                         preferred_element_type=jnp.float32)


def _tc3_body(p_ref, dis_ref, b_ref, out_ref):
    out_ref[...] = (dis_ref[:N] * (p_ref[0, :N, :] + p_ref[1, :N, :])
                    + b_ref[...])


_tc1 = pl.pallas_call(
    _tc1_body,
    out_shape=(jax.ShapeDtypeStruct((NP, 1), jnp.float32),
               jax.ShapeDtypeStruct((N, D), jnp.float32)),
)
_tc2 = pl.pallas_call(
    _tc2_body,
    out_shape=jax.ShapeDtypeStruct((N, D), jnp.float32),
)
_tc3 = pl.pallas_call(
    _tc3_body,
    out_shape=jax.ShapeDtypeStruct((N, D), jnp.float32),
)


def kernel(x, edge_index, edge_weight, W1, b1, W2, b2):
    row = edge_index[0].astype(jnp.int32)
    col = edge_index[1].astype(jnp.int32)
    ew = edge_weight.astype(jnp.float32)

    loop_idx = jnp.arange(N, dtype=jnp.int32)
    npad = E_EXT - E - N
    zpad_i = jnp.zeros((npad,), jnp.int32)
    row_e = jnp.concatenate([row, loop_idx, zpad_i]).reshape(NW, EPW)
    col_e = jnp.concatenate([col, loop_idx, zpad_i]).reshape(NW, EPW)
    ew_e = jnp.concatenate([ew, jnp.ones((N,), jnp.float32),
                            jnp.zeros((npad,), jnp.float32)]).reshape(NW, EPW)

    zeros1 = jnp.zeros((NP,), jnp.float32)

    degp = _deg_sc(col_e, ew_e, zeros1)
    dis, y1 = _tc1(degp.reshape(NC, NP, 1), x, W1)
    p1 = _prop_sc(row_e, col_e, ew_e, y1)
    y2 = _tc2(p1, dis, b1.reshape(1, D), W2)
    p2 = _prop_sc(row_e, col_e, ew_e, y2)
    out = _tc3(p2, dis, b2.reshape(1, D))
    return out
